# baseline (speedup 1.0000x reference)
import functools

import jax
import jax.numpy as jnp
from jax import lax
from jax.experimental import pallas as pl
from jax.experimental.pallas import tpu as pltpu
from jax.experimental.pallas import tpu_sc as plsc

N_NODES = 10000
N_EDGES = 320000
D = 128
H2 = 256

NB = 2000
EB = 6400

NC = 2
NS = 16
NW = NC * NS
G = 32
NGROUPS = 320
PH = 40
NPHASE = NGROUPS // PH
EPW = G * NGROUPS
IDX_PAD = NW * EPW
NG_LAST = (N_EDGES - (NW - 1) * EPW) // G
RPS = 632
RPS_LAST = N_NODES - 15 * RPS


def _edge_linear_body(ea_ref, wet_ref, be_ref, out_ref):
    out_ref[...] = (
        jnp.dot(ea_ref[...], wet_ref[...], preferred_element_type=jnp.float32)
        + be_ref[...]
    )


def _edge_linear(edge_attr, We, be):
    wet = We.T
    grid = N_EDGES // EB
    return pl.pallas_call(
        _edge_linear_body,
        grid=(grid,),
        in_specs=[
            pl.BlockSpec((EB, 16), lambda i: (i, 0)),
            pl.BlockSpec((16, D), lambda i: (0, 0)),
            pl.BlockSpec((1, D), lambda i: (0, 0)),
        ],
        out_specs=pl.BlockSpec((EB, D), lambda i: (i, 0)),
        out_shape=jax.ShapeDtypeStruct((N_EDGES, D), jnp.float32),
    )(edge_attr, wet, be.reshape(1, D))


def _sc_aggregate_body(x_hbm, srcp_hbm, dstp_hbm, e_hbm, zeros_hbm, out_hbm,
                       aggr, src_v, dst_v,
                       e0, e1, e2, e3, x0, x1, x2, x3,
                       c0, c1, c2, c3, s0, s1, s2, s3):
    cid = lax.axis_index("c")
    sid = lax.axis_index("s")
    wid = sid * NC + cid
    base = wid * EPW
    rows = pl.ds(sid * RPS, RPS)
    rows_last = pl.ds(15 * RPS, RPS_LAST)
    ebufs = (e0, e1, e2, e3)
    xbufs = (x0, x1, x2, x3)
    csems = (c0, c1, c2, c3)
    ssems = (s0, s1, s2, s3)

    @pl.when(sid < 15)
    def _():
        pltpu.sync_copy(zeros_hbm, aggr.at[rows])

    @pl.when(sid == 15)
    def _():
        pltpu.sync_copy(zeros_hbm.at[pl.ds(0, RPS_LAST)], aggr.at[rows_last])

    plsc.subcore_barrier()

    ngw = jnp.where(wid == NW - 1, NG_LAST, NGROUPS)

    def issue(j, b, ph):
        pltpu.async_copy(
            e_hbm.at[pl.ds(base + (ph * PH + j) * G, G)], ebufs[b], csems[b])
        pltpu.async_copy(x_hbm.at[src_v.at[j]], xbufs[b], csems[b])

    def wait_in(j, b, ph):
        pltpu.make_async_copy(
            e_hbm.at[pl.ds(base + (ph * PH + j) * G, G)], ebufs[b], csems[b]
        ).wait()
        pltpu.make_async_copy(x_hbm.at[src_v.at[j]], xbufs[b], csems[b]).wait()

    def wait_scatter(j, b):
        pltpu.make_async_copy(ebufs[b], aggr.at[dst_v.at[j]], ssems[b]).wait()

    for ph in range(NPHASE):
        rem = jnp.clip(ngw - ph * PH, 0, PH)

        @pl.when(rem > 0)
        def _(ph=ph, rem=rem):
            pltpu.sync_copy(srcp_hbm.at[wid, pl.ds(ph * PH, PH)], src_v)
            pltpu.sync_copy(dstp_hbm.at[wid, pl.ds(ph * PH, PH)], dst_v)
            issue(0, 0, ph)
            issue(1, 1, ph)

            def quad(i, carry, ph=ph, rem=rem):
                for b in range(4):
                    j = 4 * i + b
                    wait_in(j, b, ph)
                    eb, xb = ebufs[b], xbufs[b]

                    def rows2(r, c2, eb=eb, xb=xb):
                        for rr in range(2):
                            for k in range(D // 16):
                                s = pl.ds(k * 16, 16)
                                eb[2 * r + rr, s] = jnp.maximum(
                                    eb[2 * r + rr, s] + xb[2 * r + rr, s], 0.0)
                        return c2

                    lax.fori_loop(0, G // 2, rows2, 0)
                    pltpu.async_copy(eb, aggr.at[dst_v.at[j]], ssems[b],
                                     add=True)

                    if b >= 2:
                        wait_scatter(j - 2, b - 2)
                    else:
                        @pl.when(j >= 2)
                        def _(j=j, b=b):
                            wait_scatter(j - 2, (b + 2) % 4)

                    @pl.when(j + 2 < rem)
                    def _(j=j, b=b, ph=ph):
                        issue(j + 2, (b + 2) % 4, ph)
                return carry

            lax.fori_loop(0, rem // 4, quad, 0)
            wait_scatter(rem - 2, 2)
            wait_scatter(rem - 1, 3)

    plsc.subcore_barrier()

    @pl.when(sid < 15)
    def _():
        pltpu.sync_copy(aggr.at[rows], out_hbm.at[cid, rows])

    @pl.when(sid == 15)
    def _():
        pltpu.sync_copy(aggr.at[rows_last], out_hbm.at[cid, rows_last])


@functools.cache
def _sc_aggregate_kernel():
    return pl.kernel(
        _sc_aggregate_body,
        out_type=jax.ShapeDtypeStruct((NC, N_NODES, D), jnp.float32),
        mesh=plsc.VectorSubcoreMesh(core_axis_name="c", subcore_axis_name="s",
                                    num_cores=NC, num_subcores=NS),
        scratch_types=(
            [pltpu.VMEM_SHARED((N_NODES, D), jnp.float32)]
            + [pltpu.VMEM((PH, G), jnp.int32)] * 2
            + [pltpu.VMEM((G, D), jnp.float32)] * 8
            + [pltpu.SemaphoreType.DMA] * 8
        ),
    )


def _sc_aggregate(*args):
    return _sc_aggregate_kernel()(*args)


def _mlp1_body(x_ref, aggr_ref, w1t_ref, b1_ref, eps_ref, h1_ref, s_ref, q_ref):
    h = (1.0 + eps_ref[0]) * x_ref[...] + aggr_ref[0] + aggr_ref[1]
    t = jnp.dot(h, w1t_ref[...], preferred_element_type=jnp.float32) + b1_ref[...]
    h1_ref[...] = t
    ps = jnp.sum(t, axis=0, keepdims=True)
    pq = jnp.sum(t * t, axis=0, keepdims=True)
    i = pl.program_id(0)

    @pl.when(i == 0)
    def _():
        s_ref[...] = jnp.zeros_like(s_ref)
        q_ref[...] = jnp.zeros_like(q_ref)

    s_ref[...] += jnp.broadcast_to(ps, s_ref.shape)
    q_ref[...] += jnp.broadcast_to(pq, q_ref.shape)


def _mlp1(x, aggr, W1, b1, eps):
    w1t = W1.T
    grid = N_NODES // NB
    return pl.pallas_call(
        _mlp1_body,
        grid=(grid,),
        in_specs=[
            pl.BlockSpec((NB, D), lambda i: (i, 0)),
            pl.BlockSpec((NC, NB, D), lambda i: (0, i, 0)),
            pl.BlockSpec((D, H2), lambda i: (0, 0)),
            pl.BlockSpec((1, H2), lambda i: (0, 0)),
            pl.BlockSpec(memory_space=pltpu.SMEM),
        ],
        out_specs=[
            pl.BlockSpec((NB, H2), lambda i: (i, 0)),
            pl.BlockSpec((8, H2), lambda i: (0, 0)),
            pl.BlockSpec((8, H2), lambda i: (0, 0)),
        ],
        out_shape=[
            jax.ShapeDtypeStruct((N_NODES, H2), jnp.float32),
            jax.ShapeDtypeStruct((8, H2), jnp.float32),
            jax.ShapeDtypeStruct((8, H2), jnp.float32),
        ],
    )(x, aggr, w1t, b1.reshape(1, H2), eps.reshape(1))


def _mlp2_body(h1_ref, scale_ref, shift_ref, w2t_ref, b2_ref, h2_ref, s_ref, q_ref):
    u = jnp.maximum(h1_ref[...] * scale_ref[...] + shift_ref[...], 0.0)
    t = jnp.dot(u, w2t_ref[...], preferred_element_type=jnp.float32) + b2_ref[...]
    h2_ref[...] = t
    ps = jnp.sum(t, axis=0, keepdims=True)
    pq = jnp.sum(t * t, axis=0, keepdims=True)
    i = pl.program_id(0)

    @pl.when(i == 0)
    def _():
        s_ref[...] = jnp.zeros_like(s_ref)
        q_ref[...] = jnp.zeros_like(q_ref)

    s_ref[...] += jnp.broadcast_to(ps, s_ref.shape)
    q_ref[...] += jnp.broadcast_to(pq, q_ref.shape)


def _mlp2(h1, scale, shift, W2, b2):
    w2t = W2.T
    grid = N_NODES // NB
    return pl.pallas_call(
        _mlp2_body,
        grid=(grid,),
        in_specs=[
            pl.BlockSpec((NB, H2), lambda i: (i, 0)),
            pl.BlockSpec((1, H2), lambda i: (0, 0)),
            pl.BlockSpec((1, H2), lambda i: (0, 0)),
            pl.BlockSpec((H2, D), lambda i: (0, 0)),
            pl.BlockSpec((1, D), lambda i: (0, 0)),
        ],
        out_specs=[
            pl.BlockSpec((NB, D), lambda i: (i, 0)),
            pl.BlockSpec((8, D), lambda i: (0, 0)),
            pl.BlockSpec((8, D), lambda i: (0, 0)),
        ],
        out_shape=[
            jax.ShapeDtypeStruct((N_NODES, D), jnp.float32),
            jax.ShapeDtypeStruct((8, D), jnp.float32),
            jax.ShapeDtypeStruct((8, D), jnp.float32),
        ],
    )(h1, scale, shift, w2t, b2.reshape(1, D))


def _final_body(h2_ref, scale_ref, shift_ref, prev_ref, coef_ref, out_ref):
    u = jnp.maximum(h2_ref[...] * scale_ref[...] + shift_ref[...], 0.0)
    out_ref[...] = prev_ref[...] * coef_ref[0] + u * coef_ref[1]


def _finalize(h2, scale, shift, prev, coef):
    grid = N_NODES // NB
    return pl.pallas_call(
        _final_body,
        grid=(grid,),
        in_specs=[
            pl.BlockSpec((NB, D), lambda i: (i, 0)),
            pl.BlockSpec((1, D), lambda i: (0, 0)),
            pl.BlockSpec((1, D), lambda i: (0, 0)),
            pl.BlockSpec((NB, D), lambda i: (i, 0)),
            pl.BlockSpec(memory_space=pltpu.SMEM),
        ],
        out_specs=pl.BlockSpec((NB, D), lambda i: (i, 0)),
        out_shape=jax.ShapeDtypeStruct((N_NODES, D), jnp.float32),
    )(h2, scale, shift, prev, jnp.asarray(coef, jnp.float32).reshape(2))


def _bn_coeffs(s, q, g, b, eps=1e-5):
    m = s[0] / N_NODES
    v = q[0] / N_NODES - m * m
    inv = g * jax.lax.rsqrt(v + eps)
    scale = inv
    shift = b - m * inv
    return scale.reshape(1, -1), shift.reshape(1, -1)


def kernel(x, edge_index, edge_attr, params):
    src = edge_index[0]
    dst = edge_index[1]
    npad = IDX_PAD - N_EDGES
    srcp = jnp.concatenate([src, jnp.zeros((npad,), src.dtype)]).reshape(NW, NGROUPS, G)
    dstp = jnp.concatenate([dst, jnp.zeros((npad,), dst.dtype)]).reshape(NW, NGROUPS, G)
    zeros = jnp.zeros((RPS, D), jnp.float32)
    h = x
    Es = [_edge_linear(edge_attr, params[i]['We'], params[i]['be'])
          for i in range(3)]
    for i in range(3):
        p = params[i]
        aggr = _sc_aggregate(h, srcp, dstp, Es[i], zeros)
        h1, s1, q1 = _mlp1(h, aggr, p['W1'], p['b1'], p['eps'])
        sc1, sh1 = _bn_coeffs(s1, q1, p['g1'], p['bt1'])
        h2, s2, q2 = _mlp2(h1, sc1, sh1, p['W2'], p['b2'])
        sc2, sh2 = _bn_coeffs(s2, q2, p['gn'], p['bn'])
        coef = (1.0, 0.3) if i == 1 else (0.0, 1.0)
        h = _finalize(h2, sc2, sh2, h, coef)
    return h


# speedup vs baseline: 1.1026x; 1.1026x over previous
import functools

import jax
import jax.numpy as jnp
from jax import lax
from jax.experimental import pallas as pl
from jax.experimental.pallas import tpu as pltpu
from jax.experimental.pallas import tpu_sc as plsc

N_NODES = 10000
N_EDGES = 320000
D = 128
H2 = 256

NB = 2000
EB = 6400

NC = 2
NS = 16
NW = NC * NS
G = 64
NGROUPS = 160
PH = 40
NPHASE = NGROUPS // PH
EPW = G * NGROUPS
IDX_PAD = NW * EPW
NG_LAST = (N_EDGES - (NW - 1) * EPW) // G
RPS = 632
RPS_LAST = N_NODES - 15 * RPS

def _edge_linear_body(ea_ref, wet_ref, be_ref, out_ref):
    out_ref[...] = (
        jnp.dot(ea_ref[...], wet_ref[...], preferred_element_type=jnp.float32)
        + be_ref[...]
    )


def _edge_linear(edge_attr, We, be):
    wet = We.T
    grid = N_EDGES // EB
    return pl.pallas_call(
        _edge_linear_body,
        grid=(grid,),
        in_specs=[
            pl.BlockSpec((EB, 16), lambda i: (i, 0)),
            pl.BlockSpec((16, D), lambda i: (0, 0)),
            pl.BlockSpec((1, D), lambda i: (0, 0)),
        ],
        out_specs=pl.BlockSpec((EB, D), lambda i: (i, 0)),
        out_shape=jax.ShapeDtypeStruct((N_EDGES, D), jnp.float32),
    )(edge_attr, wet, be.reshape(1, D))


def _sc_aggregate_body(x_hbm, srcp_hbm, dstp_hbm, e_hbm, zeros_hbm, out_hbm,
                       aggr, src_v, dst_v, e0, e1, x0, x1, c0, c1):
    cid = lax.axis_index("c")
    sid = lax.axis_index("s")
    wid = sid * NC + cid
    base = wid * EPW
    rows = pl.ds(sid * RPS, RPS)
    rows_last = pl.ds(15 * RPS, RPS_LAST)
    ebufs = (e0, e1)
    xbufs = (x0, x1)
    csems = (c0, c1)

    @pl.when(sid < 15)
    def _():
        pltpu.sync_copy(zeros_hbm, aggr.at[rows])

    @pl.when(sid == 15)
    def _():
        pltpu.sync_copy(zeros_hbm.at[pl.ds(0, RPS_LAST)], aggr.at[rows_last])

    plsc.subcore_barrier()

    ngw = jnp.where(wid == NW - 1, NG_LAST, NGROUPS)

    def issue(j, b, ph):
        pltpu.async_copy(
            e_hbm.at[pl.ds(base + (ph * PH + j) * G, G)], ebufs[b], csems[b])
        pltpu.async_copy(x_hbm.at[src_v.at[j]], xbufs[b], csems[b])

    def wait_in(j, b, ph):
        pltpu.make_async_copy(
            e_hbm.at[pl.ds(base + (ph * PH + j) * G, G)], ebufs[b], csems[b]
        ).wait()
        pltpu.make_async_copy(x_hbm.at[src_v.at[j]], xbufs[b], csems[b]).wait()

    for ph in range(NPHASE):
        rem = jnp.clip(ngw - ph * PH, 0, PH)

        @pl.when(rem > 0)
        def _(ph=ph, rem=rem):
            pltpu.sync_copy(srcp_hbm.at[wid, pl.ds(ph * PH, PH)], src_v)
            pltpu.sync_copy(dstp_hbm.at[wid, pl.ds(ph * PH, PH)], dst_v)
            issue(0, 0, ph)
            issue(1, 1, ph)

            def pair(i, carry, ph=ph, rem=rem):
                for b in range(2):
                    j = 2 * i + b
                    wait_in(j, b, ph)
                    eb, xb = ebufs[b], xbufs[b]

                    def rows2(r, c2, eb=eb, xb=xb):
                        for rr in range(2):
                            for k in range(D // 16):
                                s = pl.ds(k * 16, 16)
                                eb[2 * r + rr, s] = jnp.maximum(
                                    eb[2 * r + rr, s] + xb[2 * r + rr, s], 0.0)
                        return c2

                    lax.fori_loop(0, G // 2, rows2, 0)
                    pltpu.sync_copy(eb, aggr.at[dst_v.at[j]], add=True)

                    @pl.when(j + 2 < rem)
                    def _(j=j, b=b, ph=ph):
                        issue(j + 2, b, ph)
                return carry

            lax.fori_loop(0, rem // 2, pair, 0)

    plsc.subcore_barrier()

    @pl.when(sid < 15)
    def _():
        pltpu.sync_copy(aggr.at[rows], out_hbm.at[cid, rows])

    @pl.when(sid == 15)
    def _():
        pltpu.sync_copy(aggr.at[rows_last], out_hbm.at[cid, rows_last])


@functools.cache
def _sc_aggregate_kernel():
    return pl.kernel(
        _sc_aggregate_body,
        out_type=jax.ShapeDtypeStruct((NC, N_NODES, D), jnp.float32),
        mesh=plsc.VectorSubcoreMesh(core_axis_name="c", subcore_axis_name="s",
                                    num_cores=NC, num_subcores=NS),
        scratch_types=(
            [pltpu.VMEM_SHARED((N_NODES, D), jnp.float32)]
            + [pltpu.VMEM((PH, G), jnp.int32)] * 2
            + [pltpu.VMEM((G, D), jnp.float32)] * 4
            + [pltpu.SemaphoreType.DMA] * 2
        ),
    )


def _sc_aggregate(*args):
    return _sc_aggregate_kernel()(*args)


def _mlp1_body(x_ref, aggr_ref, w1t_ref, b1_ref, eps_ref, h1_ref, s_ref, q_ref):
    h = (1.0 + eps_ref[0]) * x_ref[...] + aggr_ref[0] + aggr_ref[1]
    t = jnp.dot(h, w1t_ref[...], preferred_element_type=jnp.float32) + b1_ref[...]
    h1_ref[...] = t
    ps = jnp.sum(t, axis=0, keepdims=True)
    pq = jnp.sum(t * t, axis=0, keepdims=True)
    i = pl.program_id(0)

    @pl.when(i == 0)
    def _():
        s_ref[...] = jnp.zeros_like(s_ref)
        q_ref[...] = jnp.zeros_like(q_ref)

    s_ref[...] += jnp.broadcast_to(ps, s_ref.shape)
    q_ref[...] += jnp.broadcast_to(pq, q_ref.shape)


def _mlp1(x, aggr, W1, b1, eps):
    w1t = W1.T
    grid = N_NODES // NB
    return pl.pallas_call(
        _mlp1_body,
        grid=(grid,),
        in_specs=[
            pl.BlockSpec((NB, D), lambda i: (i, 0)),
            pl.BlockSpec((NC, NB, D), lambda i: (0, i, 0)),
            pl.BlockSpec((D, H2), lambda i: (0, 0)),
            pl.BlockSpec((1, H2), lambda i: (0, 0)),
            pl.BlockSpec(memory_space=pltpu.SMEM),
        ],
        out_specs=[
            pl.BlockSpec((NB, H2), lambda i: (i, 0)),
            pl.BlockSpec((8, H2), lambda i: (0, 0)),
            pl.BlockSpec((8, H2), lambda i: (0, 0)),
        ],
        out_shape=[
            jax.ShapeDtypeStruct((N_NODES, H2), jnp.float32),
            jax.ShapeDtypeStruct((8, H2), jnp.float32),
            jax.ShapeDtypeStruct((8, H2), jnp.float32),
        ],
    )(x, aggr, w1t, b1.reshape(1, H2), eps.reshape(1))


def _mlp2_body(h1_ref, scale_ref, shift_ref, w2t_ref, b2_ref, h2_ref, s_ref, q_ref):
    u = jnp.maximum(h1_ref[...] * scale_ref[...] + shift_ref[...], 0.0)
    t = jnp.dot(u, w2t_ref[...], preferred_element_type=jnp.float32) + b2_ref[...]
    h2_ref[...] = t
    ps = jnp.sum(t, axis=0, keepdims=True)
    pq = jnp.sum(t * t, axis=0, keepdims=True)
    i = pl.program_id(0)

    @pl.when(i == 0)
    def _():
        s_ref[...] = jnp.zeros_like(s_ref)
        q_ref[...] = jnp.zeros_like(q_ref)

    s_ref[...] += jnp.broadcast_to(ps, s_ref.shape)
    q_ref[...] += jnp.broadcast_to(pq, q_ref.shape)


def _mlp2(h1, scale, shift, W2, b2):
    w2t = W2.T
    grid = N_NODES // NB
    return pl.pallas_call(
        _mlp2_body,
        grid=(grid,),
        in_specs=[
            pl.BlockSpec((NB, H2), lambda i: (i, 0)),
            pl.BlockSpec((1, H2), lambda i: (0, 0)),
            pl.BlockSpec((1, H2), lambda i: (0, 0)),
            pl.BlockSpec((H2, D), lambda i: (0, 0)),
            pl.BlockSpec((1, D), lambda i: (0, 0)),
        ],
        out_specs=[
            pl.BlockSpec((NB, D), lambda i: (i, 0)),
            pl.BlockSpec((8, D), lambda i: (0, 0)),
            pl.BlockSpec((8, D), lambda i: (0, 0)),
        ],
        out_shape=[
            jax.ShapeDtypeStruct((N_NODES, D), jnp.float32),
            jax.ShapeDtypeStruct((8, D), jnp.float32),
            jax.ShapeDtypeStruct((8, D), jnp.float32),
        ],
    )(h1, scale, shift, w2t, b2.reshape(1, D))


def _final_body(h2_ref, scale_ref, shift_ref, prev_ref, coef_ref, out_ref):
    u = jnp.maximum(h2_ref[...] * scale_ref[...] + shift_ref[...], 0.0)
    out_ref[...] = prev_ref[...] * coef_ref[0] + u * coef_ref[1]


def _finalize(h2, scale, shift, prev, coef):
    grid = N_NODES // NB
    return pl.pallas_call(
        _final_body,
        grid=(grid,),
        in_specs=[
            pl.BlockSpec((NB, D), lambda i: (i, 0)),
            pl.BlockSpec((1, D), lambda i: (0, 0)),
            pl.BlockSpec((1, D), lambda i: (0, 0)),
            pl.BlockSpec((NB, D), lambda i: (i, 0)),
            pl.BlockSpec(memory_space=pltpu.SMEM),
        ],
        out_specs=pl.BlockSpec((NB, D), lambda i: (i, 0)),
        out_shape=jax.ShapeDtypeStruct((N_NODES, D), jnp.float32),
    )(h2, scale, shift, prev, jnp.asarray(coef, jnp.float32).reshape(2))


def _bn_coeffs(s, q, g, b, eps=1e-5):
    m = s[0] / N_NODES
    v = q[0] / N_NODES - m * m
    inv = g * jax.lax.rsqrt(v + eps)
    scale = inv
    shift = b - m * inv
    return scale.reshape(1, -1), shift.reshape(1, -1)


def kernel(x, edge_index, edge_attr, params):
    src = edge_index[0]
    dst = edge_index[1]
    npad = IDX_PAD - N_EDGES
    srcp = jnp.concatenate([src, jnp.zeros((npad,), src.dtype)]).reshape(NW, NGROUPS, G)
    dstp = jnp.concatenate([dst, jnp.zeros((npad,), dst.dtype)]).reshape(NW, NGROUPS, G)
    zeros = jnp.zeros((RPS, D), jnp.float32)
    h = x
    Es = [_edge_linear(edge_attr, params[i]['We'], params[i]['be'])
          for i in range(3)]
    for i in range(3):
        p = params[i]
        aggr = _sc_aggregate(h, srcp, dstp, Es[i], zeros)
        h1, s1, q1 = _mlp1(h, aggr, p['W1'], p['b1'], p['eps'])
        sc1, sh1 = _bn_coeffs(s1, q1, p['g1'], p['bt1'])
        h2, s2, q2 = _mlp2(h1, sc1, sh1, p['W2'], p['b2'])
        sc2, sh2 = _bn_coeffs(s2, q2, p['gn'], p['bn'])
        coef = (1.0, 0.3) if i == 1 else (0.0, 1.0)
        h = _finalize(h2, sc2, sh2, h, coef)
    return h


# speedup vs baseline: 1.1522x; 1.0451x over previous
import functools

import jax
import jax.numpy as jnp
from jax import lax
from jax.experimental import pallas as pl
from jax.experimental.pallas import tpu as pltpu
from jax.experimental.pallas import tpu_sc as plsc

N_NODES = 10000
N_EDGES = 320000
D = 128
H2 = 256

NB = 2000
EB = 6400

NC = 2
NS = 16
NW = NC * NS
G = 64
NGROUPS = 160
PH = 40
NPHASE = NGROUPS // PH
EPW = G * NGROUPS
IDX_PAD = NW * EPW
NG_LAST = (N_EDGES - (NW - 1) * EPW) // G
RPS = 632
RPS_LAST = N_NODES - 15 * RPS

def _edge_linear_body(ea_ref, wet_ref, be_ref, out_ref):
    out_ref[...] = (
        jnp.dot(ea_ref[...], wet_ref[...], preferred_element_type=jnp.float32)
        + be_ref[...]
    )


def _edge_linear(edge_attr, We, be):
    wet = We.T
    grid = N_EDGES // EB
    return pl.pallas_call(
        _edge_linear_body,
        grid=(grid,),
        in_specs=[
            pl.BlockSpec((EB, 16), lambda i: (i, 0)),
            pl.BlockSpec((16, D), lambda i: (0, 0)),
            pl.BlockSpec((1, D), lambda i: (0, 0)),
        ],
        out_specs=pl.BlockSpec((EB, D), lambda i: (i, 0)),
        out_shape=jax.ShapeDtypeStruct((N_EDGES, D), jnp.float32),
    )(edge_attr, wet, be.reshape(1, D))


def _sc_aggregate_body(x_hbm, srcp_hbm, dstp_hbm, e_hbm, zeros_hbm, out_hbm,
                       aggr, src_v, dst_v, e0, e1, x0, x1, c0, c1):
    cid = lax.axis_index("c")
    sid = lax.axis_index("s")
    wid = sid * NC + cid
    base = wid * EPW
    rows = pl.ds(sid * RPS, RPS)
    rows_last = pl.ds(15 * RPS, RPS_LAST)
    ebufs = (e0, e1)
    xbufs = (x0, x1)
    csems = (c0, c1)

    @pl.when(sid < 15)
    def _():
        pltpu.sync_copy(zeros_hbm, aggr.at[rows])

    @pl.when(sid == 15)
    def _():
        pltpu.sync_copy(zeros_hbm.at[pl.ds(0, RPS_LAST)], aggr.at[rows_last])

    plsc.subcore_barrier()

    ngw = jnp.where(wid == NW - 1, NG_LAST, NGROUPS)

    def issue_e(j, b, ph):
        pltpu.async_copy(
            e_hbm.at[pl.ds(base + (ph * PH + j) * G, G)], ebufs[b], csems[b])

    def issue_x(j, b):
        pltpu.async_copy(x_hbm.at[src_v.at[j]], xbufs[b], csems[b])

    def issue(j, b, ph):
        issue_e(j, b, ph)
        issue_x(j, b)

    def wait_in(j, b, ph):
        pltpu.make_async_copy(
            e_hbm.at[pl.ds(base + (ph * PH + j) * G, G)], ebufs[b], csems[b]
        ).wait()
        pltpu.make_async_copy(x_hbm.at[src_v.at[j]], xbufs[b], csems[b]).wait()

    for ph in range(NPHASE):
        rem = jnp.clip(ngw - ph * PH, 0, PH)

        @pl.when(rem > 0)
        def _(ph=ph, rem=rem):
            pltpu.sync_copy(srcp_hbm.at[wid, pl.ds(ph * PH, PH)], src_v)
            pltpu.sync_copy(dstp_hbm.at[wid, pl.ds(ph * PH, PH)], dst_v)
            issue(0, 0, ph)
            issue(1, 1, ph)

            def pair(i, carry, ph=ph, rem=rem):
                for b in range(2):
                    j = 2 * i + b
                    wait_in(j, b, ph)
                    eb, xb = ebufs[b], xbufs[b]

                    def rows2(r, c2, eb=eb, xb=xb):
                        for rr in range(2):
                            for k in range(D // 16):
                                s = pl.ds(k * 16, 16)
                                eb[2 * r + rr, s] = jnp.maximum(
                                    eb[2 * r + rr, s] + xb[2 * r + rr, s], 0.0)
                        return c2

                    lax.fori_loop(0, G // 2, rows2, 0)

                    @pl.when(j + 2 < rem)
                    def _(j=j, b=b):
                        issue_x(j + 2, b)

                    pltpu.sync_copy(eb, aggr.at[dst_v.at[j]], add=True)

                    @pl.when(j + 2 < rem)
                    def _(j=j, b=b, ph=ph):
                        issue_e(j + 2, b, ph)
                return carry

            lax.fori_loop(0, rem // 2, pair, 0)

    plsc.subcore_barrier()

    @pl.when(sid < 15)
    def _():
        pltpu.sync_copy(aggr.at[rows], out_hbm.at[cid, rows])

    @pl.when(sid == 15)
    def _():
        pltpu.sync_copy(aggr.at[rows_last], out_hbm.at[cid, rows_last])


@functools.cache
def _sc_aggregate_kernel():
    return pl.kernel(
        _sc_aggregate_body,
        out_type=jax.ShapeDtypeStruct((NC, N_NODES, D), jnp.float32),
        mesh=plsc.VectorSubcoreMesh(core_axis_name="c", subcore_axis_name="s",
                                    num_cores=NC, num_subcores=NS),
        scratch_types=(
            [pltpu.VMEM_SHARED((N_NODES, D), jnp.float32)]
            + [pltpu.VMEM((PH, G), jnp.int32)] * 2
            + [pltpu.VMEM((G, D), jnp.float32)] * 4
            + [pltpu.SemaphoreType.DMA] * 2
        ),
    )


def _sc_aggregate(*args):
    return _sc_aggregate_kernel()(*args)


def _mlp1_body(x_ref, aggr_ref, w1t_ref, b1_ref, eps_ref, h1_ref, s_ref, q_ref):
    h = (1.0 + eps_ref[0]) * x_ref[...] + aggr_ref[0] + aggr_ref[1]
    t = jnp.dot(h, w1t_ref[...], preferred_element_type=jnp.float32) + b1_ref[...]
    h1_ref[...] = t
    ps = jnp.sum(t, axis=0, keepdims=True)
    pq = jnp.sum(t * t, axis=0, keepdims=True)
    i = pl.program_id(0)

    @pl.when(i == 0)
    def _():
        s_ref[...] = jnp.zeros_like(s_ref)
        q_ref[...] = jnp.zeros_like(q_ref)

    s_ref[...] += jnp.broadcast_to(ps, s_ref.shape)
    q_ref[...] += jnp.broadcast_to(pq, q_ref.shape)


def _mlp1(x, aggr, W1, b1, eps):
    w1t = W1.T
    grid = N_NODES // NB
    return pl.pallas_call(
        _mlp1_body,
        grid=(grid,),
        in_specs=[
            pl.BlockSpec((NB, D), lambda i: (i, 0)),
            pl.BlockSpec((NC, NB, D), lambda i: (0, i, 0)),
            pl.BlockSpec((D, H2), lambda i: (0, 0)),
            pl.BlockSpec((1, H2), lambda i: (0, 0)),
            pl.BlockSpec(memory_space=pltpu.SMEM),
        ],
        out_specs=[
            pl.BlockSpec((NB, H2), lambda i: (i, 0)),
            pl.BlockSpec((8, H2), lambda i: (0, 0)),
            pl.BlockSpec((8, H2), lambda i: (0, 0)),
        ],
        out_shape=[
            jax.ShapeDtypeStruct((N_NODES, H2), jnp.float32),
            jax.ShapeDtypeStruct((8, H2), jnp.float32),
            jax.ShapeDtypeStruct((8, H2), jnp.float32),
        ],
    )(x, aggr, w1t, b1.reshape(1, H2), eps.reshape(1))


def _mlp2_body(h1_ref, scale_ref, shift_ref, w2t_ref, b2_ref, h2_ref, s_ref, q_ref):
    u = jnp.maximum(h1_ref[...] * scale_ref[...] + shift_ref[...], 0.0)
    t = jnp.dot(u, w2t_ref[...], preferred_element_type=jnp.float32) + b2_ref[...]
    h2_ref[...] = t
    ps = jnp.sum(t, axis=0, keepdims=True)
    pq = jnp.sum(t * t, axis=0, keepdims=True)
    i = pl.program_id(0)

    @pl.when(i == 0)
    def _():
        s_ref[...] = jnp.zeros_like(s_ref)
        q_ref[...] = jnp.zeros_like(q_ref)

    s_ref[...] += jnp.broadcast_to(ps, s_ref.shape)
    q_ref[...] += jnp.broadcast_to(pq, q_ref.shape)


def _mlp2(h1, scale, shift, W2, b2):
    w2t = W2.T
    grid = N_NODES // NB
    return pl.pallas_call(
        _mlp2_body,
        grid=(grid,),
        in_specs=[
            pl.BlockSpec((NB, H2), lambda i: (i, 0)),
            pl.BlockSpec((1, H2), lambda i: (0, 0)),
            pl.BlockSpec((1, H2), lambda i: (0, 0)),
            pl.BlockSpec((H2, D), lambda i: (0, 0)),
            pl.BlockSpec((1, D), lambda i: (0, 0)),
        ],
        out_specs=[
            pl.BlockSpec((NB, D), lambda i: (i, 0)),
            pl.BlockSpec((8, D), lambda i: (0, 0)),
            pl.BlockSpec((8, D), lambda i: (0, 0)),
        ],
        out_shape=[
            jax.ShapeDtypeStruct((N_NODES, D), jnp.float32),
            jax.ShapeDtypeStruct((8, D), jnp.float32),
            jax.ShapeDtypeStruct((8, D), jnp.float32),
        ],
    )(h1, scale, shift, w2t, b2.reshape(1, D))


def _final_body(h2_ref, scale_ref, shift_ref, prev_ref, coef_ref, out_ref):
    u = jnp.maximum(h2_ref[...] * scale_ref[...] + shift_ref[...], 0.0)
    out_ref[...] = prev_ref[...] * coef_ref[0] + u * coef_ref[1]


def _finalize(h2, scale, shift, prev, coef):
    grid = N_NODES // NB
    return pl.pallas_call(
        _final_body,
        grid=(grid,),
        in_specs=[
            pl.BlockSpec((NB, D), lambda i: (i, 0)),
            pl.BlockSpec((1, D), lambda i: (0, 0)),
            pl.BlockSpec((1, D), lambda i: (0, 0)),
            pl.BlockSpec((NB, D), lambda i: (i, 0)),
            pl.BlockSpec(memory_space=pltpu.SMEM),
        ],
        out_specs=pl.BlockSpec((NB, D), lambda i: (i, 0)),
        out_shape=jax.ShapeDtypeStruct((N_NODES, D), jnp.float32),
    )(h2, scale, shift, prev, jnp.asarray(coef, jnp.float32).reshape(2))


def _bn_coeffs(s, q, g, b, eps=1e-5):
    m = s[0] / N_NODES
    v = q[0] / N_NODES - m * m
    inv = g * jax.lax.rsqrt(v + eps)
    scale = inv
    shift = b - m * inv
    return scale.reshape(1, -1), shift.reshape(1, -1)


def kernel(x, edge_index, edge_attr, params):
    src = edge_index[0]
    dst = edge_index[1]
    npad = IDX_PAD - N_EDGES
    srcp = jnp.concatenate([src, jnp.zeros((npad,), src.dtype)]).reshape(NW, NGROUPS, G)
    dstp = jnp.concatenate([dst, jnp.zeros((npad,), dst.dtype)]).reshape(NW, NGROUPS, G)
    zeros = jnp.zeros((RPS, D), jnp.float32)
    h = x
    Es = [_edge_linear(edge_attr, params[i]['We'], params[i]['be'])
          for i in range(3)]
    for i in range(3):
        p = params[i]
        aggr = _sc_aggregate(h, srcp, dstp, Es[i], zeros)
        h1, s1, q1 = _mlp1(h, aggr, p['W1'], p['b1'], p['eps'])
        sc1, sh1 = _bn_coeffs(s1, q1, p['g1'], p['bt1'])
        h2, s2, q2 = _mlp2(h1, sc1, sh1, p['W2'], p['b2'])
        sc2, sh2 = _bn_coeffs(s2, q2, p['gn'], p['bn'])
        coef = (1.0, 0.3) if i == 1 else (0.0, 1.0)
        h = _finalize(h2, sc2, sh2, h, coef)
    return h
